# baseline (speedup 1.0000x reference)
import functools
import numpy as np
import jax
import jax.numpy as jnp
from jax import lax
from jax.experimental import pallas as pl
from jax.experimental.pallas import tpu as pltpu
from jax.experimental.pallas import tpu_sc as plsc

N = 10000
E = 320000
H = 128
RSQRT_H = float(1.0 / np.sqrt(128.0))

NC = 2
NS = 16
NW = NC * NS
C = 64
EP = 321536
EPW = EP // NW
NCHUNK = EPW // C
NP = 10112
NPT = NP // NS
HQ = 144
DPAD = 10104

@functools.cache
def _mesh():
    return plsc.VectorSubcoreMesh(core_axis_name="c", subcore_axis_name="s",
                                  num_cores=NC, num_subcores=NS)

_CPARAMS = None


def _cparams():
    return pltpu.CompilerParams(use_tc_tiling_on_sc=False,
                                needs_layout_passes=False)


def _conv_edge_body(src_h, dst_h, q_h, kv_h, ew_h, z128_h, z1_h,
                    acc_o, den_o,
                    sidx3, didx3, qb, kvb, ewb2, exb2,
                    acc_sp, den_sp,
                    sem_q, sem_kv, sem_ew, sem_sc, sem_ix):
    cid = lax.axis_index("c")
    sid = lax.axis_index("s")
    wid = cid * NS + sid
    row0 = sid * NPT
    pltpu.sync_copy(z128_h.at[pl.ds(row0, NPT)], acc_sp.at[pl.ds(row0, NPT)])
    pltpu.sync_copy(z1_h.at[pl.ds(row0, NPT)], den_sp.at[pl.ds(row0, NPT)])
    plsc.subcore_barrier()

    e0 = wid * EPW
    pltpu.sync_copy(src_h.at[pl.ds(e0, C)], sidx3.at[0])
    pltpu.sync_copy(dst_h.at[pl.ds(e0, C)], didx3.at[0])
    pltpu.async_copy(q_h.at[didx3.at[0]], qb, sem_q)
    pltpu.async_copy(kv_h.at[sidx3.at[0]], kvb, sem_kv)
    pltpu.async_copy(ew_h.at[pl.ds(e0, C)], ewb2.at[0], sem_ew)
    pltpu.async_copy(src_h.at[pl.ds(e0 + C, C)], sidx3.at[1], sem_ix)
    pltpu.async_copy(dst_h.at[pl.ds(e0 + C, C)], didx3.at[1], sem_ix)

    lanes = lax.iota(jnp.int32, 16)
    perms = [lanes ^ sh for sh in (8, 4, 2, 1)]

    def compute(cur):
        def group(g, c2):
            av = jnp.zeros((16,), jnp.float32)
            for j in range(16):
                e = g * 16 + j
                acc = jnp.zeros((16,), jnp.float32)
                for t in range(8):
                    sl = pl.ds(t * 16, 16)
                    acc = acc + qb[e, sl] * (kvb[e, sl] + ewb2[cur, e, sl])
                acc = acc + qb[e, pl.ds(128, 16)]
                for perm in perms:
                    acc = acc + acc.at[perm].get(mode="promise_in_bounds")
                av = jnp.where(lanes == j, acc, av)
            exv = jnp.exp(av)
            exb2[cur, pl.ds(g * 16, 16)] = exv
            for j in range(16):
                e = g * 16 + j
                x = exv[j]
                for t in range(8):
                    ewb2[cur, e, pl.ds(t * 16, 16)] = x * (
                        kvb[e, pl.ds(128 + t * 16, 16)]
                        + ewb2[cur, e, pl.ds(t * 16, 16)])
            return c2
        lax.fori_loop(0, C // 16, group, 0, unroll=False)

    def scat_waits(par):
        pltpu.make_async_copy(ewb2.at[par], acc_sp.at[didx3.at[0]], sem_sc).wait()
        pltpu.make_async_copy(exb2.at[par], den_sp.at[didx3.at[0]], sem_sc).wait()

    def iter_body(ci, carry):
        cur = lax.rem(ci, 2)
        nxt = 1 - cur
        ix = lax.rem(ci, 3)
        base = e0 + ci * C
        pltpu.make_async_copy(q_h.at[didx3.at[ix]], qb, sem_q).wait()
        pltpu.make_async_copy(kv_h.at[sidx3.at[ix]], kvb, sem_kv).wait()
        pltpu.make_async_copy(ew_h.at[pl.ds(base, C)], ewb2.at[cur], sem_ew).wait()
        compute(cur)
        pltpu.async_copy(ewb2.at[cur], acc_sp.at[didx3.at[ix]], sem_sc, add=True)
        pltpu.async_copy(exb2.at[cur], den_sp.at[didx3.at[ix]], sem_sc, add=True)

        @pl.when(ci + 1 < NCHUNK)
        def _():
            nix = lax.rem(ci + 1, 3)
            nbase = base + C
            pltpu.make_async_copy(src_h.at[pl.ds(nbase, C)], sidx3.at[nix], sem_ix).wait()
            pltpu.make_async_copy(dst_h.at[pl.ds(nbase, C)], didx3.at[nix], sem_ix).wait()
            pltpu.async_copy(q_h.at[didx3.at[nix]], qb, sem_q)
            pltpu.async_copy(kv_h.at[sidx3.at[nix]], kvb, sem_kv)

            @pl.when(ci > 0)
            def _():
                scat_waits(nxt)

            pltpu.async_copy(ew_h.at[pl.ds(nbase, C)], ewb2.at[nxt], sem_ew)

            @pl.when(ci + 2 < NCHUNK)
            def _():
                pix = lax.rem(ci + 2, 3)
                pbase = base + 2 * C
                pltpu.async_copy(src_h.at[pl.ds(pbase, C)], sidx3.at[pix], sem_ix)
                pltpu.async_copy(dst_h.at[pl.ds(pbase, C)], didx3.at[pix], sem_ix)

        return carry

    lax.fori_loop(0, NCHUNK, iter_body, 0, unroll=False)
    scat_waits(0)
    scat_waits(1)
    plsc.subcore_barrier()
    pltpu.sync_copy(acc_sp.at[pl.ds(row0, NPT)], acc_o.at[cid, pl.ds(row0, NPT)])
    pltpu.sync_copy(den_sp.at[pl.ds(row0, NPT)], den_o.at[cid, pl.ds(row0, NPT)])


@functools.cache
def _conv_edge_pass():
    return pl.kernel(
        _conv_edge_body,
        out_type=[jax.ShapeDtypeStruct((NC, NP, H), jnp.float32),
                  jax.ShapeDtypeStruct((NC, NP), jnp.float32)],
        mesh=_mesh(),
        compiler_params=_cparams(),
        scratch_types=[
            pltpu.VMEM((3, C), jnp.int32),
            pltpu.VMEM((3, C), jnp.int32),
            pltpu.VMEM((C, HQ), jnp.float32),
            pltpu.VMEM((C, 2 * H), jnp.float32),
            pltpu.VMEM((2, C, H), jnp.float32),
            pltpu.VMEM((2, C), jnp.float32),
            pltpu.VMEM_SHARED((NP, H), jnp.float32),
            pltpu.VMEM_SHARED((NP,), jnp.float32),
            pltpu.SemaphoreType.DMA,
            pltpu.SemaphoreType.DMA,
            pltpu.SemaphoreType.DMA,
            pltpu.SemaphoreType.DMA,
            pltpu.SemaphoreType.DMA,
        ],
    )


def _eu_body(src_h, dst_h, a_h, b_h, t_h, wv_h,
             ea_o, rm_o, dv_o,
             sidx3, didx3, ab2, bb2, tb2, dvb2, rmb, wvb,
             sem_a, sem_b, sem_t, sem_st, sem_ix):
    cid = lax.axis_index("c")
    sid = lax.axis_index("s")
    wid = cid * NS + sid
    pltpu.sync_copy(wv_h, wvb)
    e0 = wid * EPW
    pltpu.sync_copy(src_h.at[pl.ds(e0, C)], sidx3.at[0])
    pltpu.sync_copy(dst_h.at[pl.ds(e0, C)], didx3.at[0])
    pltpu.async_copy(a_h.at[sidx3.at[0]], ab2.at[0], sem_a)
    pltpu.async_copy(b_h.at[didx3.at[0]], bb2.at[0], sem_b)
    pltpu.async_copy(t_h.at[pl.ds(e0, C)], tb2.at[0], sem_t)
    pltpu.async_copy(src_h.at[pl.ds(e0 + C, C)], sidx3.at[1], sem_ix)
    pltpu.async_copy(dst_h.at[pl.ds(e0 + C, C)], didx3.at[1], sem_ix)

    lanes = lax.iota(jnp.int32, 16)
    perms = [lanes ^ sh for sh in (8, 4, 2, 1)]

    def st_waits(par, base):
        pltpu.make_async_copy(tb2.at[par], ea_o.at[pl.ds(base, C)], sem_st).wait()
        pltpu.make_async_copy(dvb2.at[par], dv_o.at[pl.ds(base, C)], sem_st).wait()

    def iter_body(ci, rm):
        cur = lax.rem(ci, 2)
        nxt = 1 - cur
        ix = lax.rem(ci, 3)
        base = e0 + ci * C
        pltpu.make_async_copy(a_h.at[sidx3.at[ix]], ab2.at[cur], sem_a).wait()
        pltpu.make_async_copy(b_h.at[didx3.at[ix]], bb2.at[cur], sem_b).wait()
        pltpu.make_async_copy(t_h.at[pl.ds(base, C)], tb2.at[cur], sem_t).wait()

        def group(g, rm2):
            dvv = jnp.zeros((16,), jnp.float32)
            for j in range(16):
                e = g * 16 + j
                sq = jnp.zeros((16,), jnp.float32)
                dv = jnp.zeros((16,), jnp.float32)
                for t in range(8):
                    sl = pl.ds(t * 16, 16)
                    z = ab2[cur, e, sl] + bb2[cur, e, sl] + tb2[cur, e, sl]
                    z = jnp.where(z > 0, z, z * jnp.float32(0.01))
                    tb2[cur, e, sl] = z
                    sq = sq + z * z
                    dv = dv + z * wvb[sl]
                for perm in perms:
                    sq = sq + sq.at[perm].get(mode="promise_in_bounds")
                    dv = dv + dv.at[perm].get(mode="promise_in_bounds")
                rm2 = jnp.maximum(rm2, sq)
                dvv = jnp.where(lanes == j, dv, dvv)
            dvb2[cur, pl.ds(g * 16, 16)] = dvv
            return rm2

        rm = lax.fori_loop(0, C // 16, group, rm, unroll=False)
        pltpu.async_copy(tb2.at[cur], ea_o.at[pl.ds(base, C)], sem_st)
        pltpu.async_copy(dvb2.at[cur], dv_o.at[pl.ds(base, C)], sem_st)

        @pl.when(ci + 1 < NCHUNK)
        def _():
            nix = lax.rem(ci + 1, 3)
            nbase = base + C
            pltpu.make_async_copy(src_h.at[pl.ds(nbase, C)], sidx3.at[nix], sem_ix).wait()
            pltpu.make_async_copy(dst_h.at[pl.ds(nbase, C)], didx3.at[nix], sem_ix).wait()
            pltpu.async_copy(a_h.at[sidx3.at[nix]], ab2.at[nxt], sem_a)
            pltpu.async_copy(b_h.at[didx3.at[nix]], bb2.at[nxt], sem_b)

            @pl.when(ci > 0)
            def _():
                st_waits(nxt, base - C)

            pltpu.async_copy(t_h.at[pl.ds(nbase, C)], tb2.at[nxt], sem_t)

            @pl.when(ci + 2 < NCHUNK)
            def _():
                pix = lax.rem(ci + 2, 3)
                pbase = base + 2 * C
                pltpu.async_copy(src_h.at[pl.ds(pbase, C)], sidx3.at[pix], sem_ix)
                pltpu.async_copy(dst_h.at[pl.ds(pbase, C)], didx3.at[pix], sem_ix)

        return rm

    rm = lax.fori_loop(0, NCHUNK, iter_body, jnp.zeros((16,), jnp.float32),
                       unroll=False)
    st_waits(0, e0)
    st_waits(1, e0)
    rmb[...] = rm
    pltpu.sync_copy(rmb, rm_o.at[cid, sid])


@functools.cache
def _eu_pass():
    return pl.kernel(
        _eu_body,
        out_type=[jax.ShapeDtypeStruct((EP, H), jnp.float32),
                  jax.ShapeDtypeStruct((NC, NS, 16), jnp.float32),
                  jax.ShapeDtypeStruct((EP,), jnp.float32)],
        mesh=_mesh(),
        compiler_params=_cparams(),
        scratch_types=[
            pltpu.VMEM((3, C), jnp.int32),
            pltpu.VMEM((3, C), jnp.int32),
            pltpu.VMEM((2, C, H), jnp.float32),
            pltpu.VMEM((2, C, H), jnp.float32),
            pltpu.VMEM((2, C, H), jnp.float32),
            pltpu.VMEM((2, C), jnp.float32),
            pltpu.VMEM((16,), jnp.float32),
            pltpu.VMEM((H,), jnp.float32),
            pltpu.SemaphoreType.DMA,
            pltpu.SemaphoreType.DMA,
            pltpu.SemaphoreType.DMA,
            pltpu.SemaphoreType.DMA,
            pltpu.SemaphoreType.DMA,
        ],
    )


def _final_body(src_h, dst_h, g1_h, g2_h, dv_h, out_o,
                sidx, didx, g1b, g2b, dvb, ob, sem):
    cid = lax.axis_index("c")
    sid = lax.axis_index("s")
    wid = cid * NS + sid

    def chunk(ci, carry):
        base = wid * EPW + ci * C
        pltpu.sync_copy(src_h.at[pl.ds(base, C)], sidx)
        pltpu.sync_copy(dst_h.at[pl.ds(base, C)], didx)
        cp_a = pltpu.async_copy(g1_h.at[sidx], g1b, sem)
        cp_b = pltpu.async_copy(g2_h.at[didx], g2b, sem)
        cp_d = pltpu.async_copy(dv_h.at[pl.ds(base, C)], dvb, sem)
        cp_a.wait()
        cp_b.wait()
        cp_d.wait()

        def group(g, c2):
            sl = pl.ds(g * 16, 16)
            z = g1b[sl] + g2b[sl] + dvb[sl]
            ob[sl] = 1.0 / (1.0 + jnp.exp(-z))
            return c2

        lax.fori_loop(0, C // 16, group, 0, unroll=False)
        pltpu.sync_copy(ob, out_o.at[pl.ds(base, C)])
        return carry

    lax.fori_loop(0, NCHUNK, chunk, 0, unroll=False)


@functools.cache
def _final_pass():
    return pl.kernel(
        _final_body,
        out_type=jax.ShapeDtypeStruct((EP,), jnp.float32),
        mesh=_mesh(),
        compiler_params=_cparams(),
        scratch_types=[
            pltpu.VMEM((C,), jnp.int32),
            pltpu.VMEM((C,), jnp.int32),
            pltpu.VMEM((C,), jnp.float32),
            pltpu.VMEM((C,), jnp.float32),
            pltpu.VMEM((C,), jnp.float32),
            pltpu.VMEM((C,), jnp.float32),
            pltpu.SemaphoreType.DMA,
        ],
    )


def _pad_nodes(x, width):
    out = jnp.zeros((NP, width), jnp.float32)
    return out.at[:N].set(x)


def _conv(h, src, dst, eaW, AW, p):
    q = h @ p["Wq"] + p["bq"]
    k = h @ p["Wk"] + p["bk"]
    v = h @ p["Wv"] + p["bv"]
    kv = _pad_nodes(jnp.concatenate([k, v], axis=1), 2 * H)
    K = jnp.max(jnp.linalg.norm(k, axis=1))
    s = jnp.linalg.norm(q, axis=1) * (K + AW) * RSQRT_H
    qpad = _pad_nodes(
        jnp.concatenate([q * RSQRT_H, -s[:, None],
                         jnp.zeros((N, HQ - H - 1), jnp.float32)], axis=1), HQ)
    z128 = jnp.zeros((NP, H), jnp.float32)
    z1 = jnp.zeros((NP,), jnp.float32)
    acc, den = _conv_edge_pass()(src, dst, qpad, kv, eaW, z128, z1)
    num = acc[0, :N] + acc[1, :N]
    denom = den[0, :N] + den[1, :N]
    out = jnp.where(denom[:, None] > 0, num / denom[:, None], 0.0)
    return out + h @ p["Ws"] + p["bs"]


def kernel(x, edge_attr, edge_index, params):
    p = params
    src = jnp.concatenate([edge_index[0], jnp.zeros((EP - E,), jnp.int32)])
    dst = jnp.concatenate([edge_index[1],
                           jnp.full((EP - E,), DPAD, jnp.int32)])
    eap = jnp.zeros((EP, 16), jnp.float32).at[:E].set(edge_attr)
    h = x @ p["node_W"] + p["node_b"]

    c1 = p["conv1"]
    eaW1 = eap @ (p["edge_W"] @ c1["We"]) + p["edge_b"] @ c1["We"]
    AW1 = jnp.max(jnp.linalg.norm(eaW1[:E], axis=1))
    h = jax.nn.leaky_relu(_conv(h, src, dst, eaW1, AW1, c1))

    w = p["eu3_W"] @ p["fc_W"]
    c = p["eu3_b"] @ p["fc_W"] + p["fc_b"]

    A1 = _pad_nodes(h @ p["eu1_W"][:H], H)
    B1 = _pad_nodes(h @ p["eu1_W"][H:2 * H], H)
    T1 = eap @ (p["edge_W"] @ p["eu1_W"][2 * H:]) + (
        p["edge_b"] @ p["eu1_W"][2 * H:] + p["eu1_b"])
    ea2, rm1, _ = _eu_pass()(src, dst, A1, B1, T1, jnp.zeros((H,), jnp.float32))
    Aea2 = jnp.sqrt(jnp.max(rm1))

    c2 = p["conv2"]
    eaW2 = ea2 @ c2["We"]
    AW2 = Aea2 * jnp.linalg.norm(c2["We"])
    h = jax.nn.leaky_relu(_conv(h, src, dst, eaW2, AW2, c2))

    A2 = _pad_nodes(h @ p["eu2_W"][:H], H)
    B2 = _pad_nodes(h @ p["eu2_W"][H:2 * H], H)
    T2 = ea2 @ p["eu2_W"][2 * H:] + p["eu2_b"]
    ea3, rm2, dvec = _eu_pass()(src, dst, A2, B2, T2, w[2 * H:, 0])
    Aea3 = jnp.sqrt(jnp.max(rm2))

    c3 = p["conv3"]
    eaW3 = ea3 @ c3["We"]
    AW3 = Aea3 * jnp.linalg.norm(c3["We"])
    h = _conv(h, src, dst, eaW3, AW3, c3)

    g1 = jnp.zeros((NP,), jnp.float32).at[:N].set((h @ w[:H])[:, 0])
    g2 = jnp.zeros((NP,), jnp.float32).at[:N].set((h @ w[H:2 * H])[:, 0] + c[0])
    out = _final_pass()(src, dst, g1, g2, dvec)
    return out[:E, None]


# speedup vs baseline: 1.1062x; 1.1062x over previous
import functools
import numpy as np
import jax
import jax.numpy as jnp
from jax import lax
from jax.experimental import pallas as pl
from jax.experimental.pallas import tpu as pltpu
from jax.experimental.pallas import tpu_sc as plsc

N = 10000
E = 320000
H = 128
RSQRT_H = float(1.0 / np.sqrt(128.0))

NC = 2
NS = 16
NW = NC * NS
C = 64
EP = 321536
EPW = EP // NW
NCHUNK = EPW // C
NP = 10112
NPT = NP // NS
HQ = 144
DPAD = 10104

@functools.cache
def _mesh():
    return plsc.VectorSubcoreMesh(core_axis_name="c", subcore_axis_name="s",
                                  num_cores=NC, num_subcores=NS)

_CPARAMS = None


def _cparams():
    return pltpu.CompilerParams(use_tc_tiling_on_sc=False,
                                needs_layout_passes=False)


CV = 80
EPW_CV = E // NW
NCHUNK_CV = EPW_CV // CV


def _conv_edge_body(src_h, dst_h, q_h, kv_h, ew_h, z128_h, z1_h,
                    acc_o, den_o,
                    sidx2, didx2, qb, kvb, ewb, exb,
                    acc_sp, den_sp,
                    sem_q, sem_kv, sem_ew, sem_sc, sem_ix):
    cid = lax.axis_index("c")
    sid = lax.axis_index("s")
    wid = cid * NS + sid
    row0 = sid * NPT
    pltpu.sync_copy(z128_h.at[pl.ds(row0, NPT)], acc_sp.at[pl.ds(row0, NPT)])
    pltpu.sync_copy(z1_h.at[pl.ds(row0, NPT)], den_sp.at[pl.ds(row0, NPT)])
    plsc.subcore_barrier()

    e0 = wid * EPW_CV
    pltpu.sync_copy(src_h.at[pl.ds(e0, CV)], sidx2.at[0])
    pltpu.sync_copy(dst_h.at[pl.ds(e0, CV)], didx2.at[0])
    pltpu.async_copy(q_h.at[didx2.at[0]], qb, sem_q)
    pltpu.async_copy(kv_h.at[sidx2.at[0]], kvb, sem_kv)
    pltpu.async_copy(ew_h.at[pl.ds(e0, CV)], ewb, sem_ew)
    pltpu.async_copy(src_h.at[pl.ds(e0 + CV, CV)], sidx2.at[1], sem_ix)
    pltpu.async_copy(dst_h.at[pl.ds(e0 + CV, CV)], didx2.at[1], sem_ix)

    lanes = lax.iota(jnp.int32, 16)
    perms = [lanes ^ sh for sh in (8, 4, 2, 1)]

    def compute():
        def group(g, c2):
            av = jnp.zeros((16,), jnp.float32)
            for j in range(16):
                e = g * 16 + j
                acc = jnp.zeros((16,), jnp.float32)
                for t in range(8):
                    sl = pl.ds(t * 16, 16)
                    acc = acc + qb[e, sl] * (kvb[e, sl] + ewb[e, sl])
                acc = acc + qb[e, pl.ds(128, 16)]
                for perm in perms:
                    acc = acc + acc.at[perm].get(mode="promise_in_bounds")
                av = jnp.where(lanes == j, acc, av)
            exv = jnp.exp(av)
            exb[pl.ds(g * 16, 16)] = exv
            for j in range(16):
                e = g * 16 + j
                x = exv[j]
                for t in range(8):
                    ewb[e, pl.ds(t * 16, 16)] = x * (
                        kvb[e, pl.ds(128 + t * 16, 16)] + ewb[e, pl.ds(t * 16, 16)])
            return c2
        lax.fori_loop(0, CV // 16, group, 0, unroll=False)

    def iter_body(ci, carry):
        cur = lax.rem(ci, 2)
        nxt = 1 - cur
        base = e0 + ci * CV
        pltpu.make_async_copy(q_h.at[didx2.at[cur]], qb, sem_q).wait()
        pltpu.make_async_copy(kv_h.at[sidx2.at[cur]], kvb, sem_kv).wait()
        pltpu.make_async_copy(ew_h.at[pl.ds(base, CV)], ewb, sem_ew).wait()
        compute()
        cp_acc = pltpu.async_copy(ewb, acc_sp.at[didx2.at[cur]], sem_sc, add=True)
        cp_den = pltpu.async_copy(exb, den_sp.at[didx2.at[cur]], sem_sc, add=True)
        nbase = e0 + (ci + 1) * CV

        @pl.when(ci + 1 < NCHUNK_CV)
        def _():
            pltpu.make_async_copy(src_h.at[pl.ds(nbase, CV)], sidx2.at[nxt], sem_ix).wait()
            pltpu.make_async_copy(dst_h.at[pl.ds(nbase, CV)], didx2.at[nxt], sem_ix).wait()
            pltpu.async_copy(q_h.at[didx2.at[nxt]], qb, sem_q)
            pltpu.async_copy(kv_h.at[sidx2.at[nxt]], kvb, sem_kv)

        cp_acc.wait()
        cp_den.wait()

        @pl.when(ci + 1 < NCHUNK_CV)
        def _():
            pltpu.async_copy(ew_h.at[pl.ds(nbase, CV)], ewb, sem_ew)

        @pl.when(ci + 2 < NCHUNK_CV)
        def _():
            pbase = e0 + (ci + 2) * CV
            pltpu.async_copy(src_h.at[pl.ds(pbase, CV)], sidx2.at[cur], sem_ix)
            pltpu.async_copy(dst_h.at[pl.ds(pbase, CV)], didx2.at[cur], sem_ix)

        return carry

    lax.fori_loop(0, NCHUNK_CV, iter_body, 0, unroll=False)
    plsc.subcore_barrier()
    pltpu.sync_copy(acc_sp.at[pl.ds(row0, NPT)], acc_o.at[cid, pl.ds(row0, NPT)])
    pltpu.sync_copy(den_sp.at[pl.ds(row0, NPT)], den_o.at[cid, pl.ds(row0, NPT)])


@functools.cache
def _conv_edge_pass():
    return pl.kernel(
        _conv_edge_body,
        out_type=[jax.ShapeDtypeStruct((NC, NP, H), jnp.float32),
                  jax.ShapeDtypeStruct((NC, NP), jnp.float32)],
        mesh=_mesh(),
        compiler_params=_cparams(),
        scratch_types=[
            pltpu.VMEM((2, CV), jnp.int32),
            pltpu.VMEM((2, CV), jnp.int32),
            pltpu.VMEM((CV, HQ), jnp.float32),
            pltpu.VMEM((CV, 2 * H), jnp.float32),
            pltpu.VMEM((CV, H), jnp.float32),
            pltpu.VMEM((CV,), jnp.float32),
            pltpu.VMEM_SHARED((NP, H), jnp.float32),
            pltpu.VMEM_SHARED((NP,), jnp.float32),
            pltpu.SemaphoreType.DMA,
            pltpu.SemaphoreType.DMA,
            pltpu.SemaphoreType.DMA,
            pltpu.SemaphoreType.DMA,
            pltpu.SemaphoreType.DMA,
        ],
    )


def _eu_body(src_h, dst_h, a_h, b_h, t_h, wv_h,
             ea_o, rm_o, dv_o,
             sidx3, didx3, ab2, bb2, tb2, dvb2, rmb, wvb,
             sem_a, sem_b, sem_t, sem_st, sem_ix):
    cid = lax.axis_index("c")
    sid = lax.axis_index("s")
    wid = cid * NS + sid
    pltpu.sync_copy(wv_h, wvb)
    e0 = wid * EPW
    pltpu.sync_copy(src_h.at[pl.ds(e0, C)], sidx3.at[0])
    pltpu.sync_copy(dst_h.at[pl.ds(e0, C)], didx3.at[0])
    pltpu.async_copy(a_h.at[sidx3.at[0]], ab2.at[0], sem_a)
    pltpu.async_copy(b_h.at[didx3.at[0]], bb2.at[0], sem_b)
    pltpu.async_copy(t_h.at[pl.ds(e0, C)], tb2.at[0], sem_t)
    pltpu.async_copy(src_h.at[pl.ds(e0 + C, C)], sidx3.at[1], sem_ix)
    pltpu.async_copy(dst_h.at[pl.ds(e0 + C, C)], didx3.at[1], sem_ix)

    lanes = lax.iota(jnp.int32, 16)
    perms = [lanes ^ sh for sh in (8, 4, 2, 1)]

    def st_waits(par, base):
        pltpu.make_async_copy(tb2.at[par], ea_o.at[pl.ds(base, C)], sem_st).wait()
        pltpu.make_async_copy(dvb2.at[par], dv_o.at[pl.ds(base, C)], sem_st).wait()

    def iter_body(ci, rm):
        cur = lax.rem(ci, 2)
        nxt = 1 - cur
        ix = lax.rem(ci, 3)
        base = e0 + ci * C
        pltpu.make_async_copy(a_h.at[sidx3.at[ix]], ab2.at[cur], sem_a).wait()
        pltpu.make_async_copy(b_h.at[didx3.at[ix]], bb2.at[cur], sem_b).wait()
        pltpu.make_async_copy(t_h.at[pl.ds(base, C)], tb2.at[cur], sem_t).wait()

        def group(g, rm2):
            dvv = jnp.zeros((16,), jnp.float32)
            for j in range(16):
                e = g * 16 + j
                sq = jnp.zeros((16,), jnp.float32)
                dv = jnp.zeros((16,), jnp.float32)
                for t in range(8):
                    sl = pl.ds(t * 16, 16)
                    z = ab2[cur, e, sl] + bb2[cur, e, sl] + tb2[cur, e, sl]
                    z = jnp.where(z > 0, z, z * jnp.float32(0.01))
                    tb2[cur, e, sl] = z
                    sq = sq + z * z
                    dv = dv + z * wvb[sl]
                for perm in perms:
                    sq = sq + sq.at[perm].get(mode="promise_in_bounds")
                    dv = dv + dv.at[perm].get(mode="promise_in_bounds")
                rm2 = jnp.maximum(rm2, sq)
                dvv = jnp.where(lanes == j, dv, dvv)
            dvb2[cur, pl.ds(g * 16, 16)] = dvv
            return rm2

        rm = lax.fori_loop(0, C // 16, group, rm, unroll=False)
        pltpu.async_copy(tb2.at[cur], ea_o.at[pl.ds(base, C)], sem_st)
        pltpu.async_copy(dvb2.at[cur], dv_o.at[pl.ds(base, C)], sem_st)

        @pl.when(ci + 1 < NCHUNK)
        def _():
            nix = lax.rem(ci + 1, 3)
            nbase = base + C
            pltpu.make_async_copy(src_h.at[pl.ds(nbase, C)], sidx3.at[nix], sem_ix).wait()
            pltpu.make_async_copy(dst_h.at[pl.ds(nbase, C)], didx3.at[nix], sem_ix).wait()
            pltpu.async_copy(a_h.at[sidx3.at[nix]], ab2.at[nxt], sem_a)
            pltpu.async_copy(b_h.at[didx3.at[nix]], bb2.at[nxt], sem_b)

            @pl.when(ci > 0)
            def _():
                st_waits(nxt, base - C)

            pltpu.async_copy(t_h.at[pl.ds(nbase, C)], tb2.at[nxt], sem_t)

            @pl.when(ci + 2 < NCHUNK)
            def _():
                pix = lax.rem(ci + 2, 3)
                pbase = base + 2 * C
                pltpu.async_copy(src_h.at[pl.ds(pbase, C)], sidx3.at[pix], sem_ix)
                pltpu.async_copy(dst_h.at[pl.ds(pbase, C)], didx3.at[pix], sem_ix)

        return rm

    rm = lax.fori_loop(0, NCHUNK, iter_body, jnp.zeros((16,), jnp.float32),
                       unroll=False)
    st_waits(0, e0)
    st_waits(1, e0)
    rmb[...] = rm
    pltpu.sync_copy(rmb, rm_o.at[cid, sid])


@functools.cache
def _eu_pass():
    return pl.kernel(
        _eu_body,
        out_type=[jax.ShapeDtypeStruct((EP, H), jnp.float32),
                  jax.ShapeDtypeStruct((NC, NS, 16), jnp.float32),
                  jax.ShapeDtypeStruct((EP,), jnp.float32)],
        mesh=_mesh(),
        compiler_params=_cparams(),
        scratch_types=[
            pltpu.VMEM((3, C), jnp.int32),
            pltpu.VMEM((3, C), jnp.int32),
            pltpu.VMEM((2, C, H), jnp.float32),
            pltpu.VMEM((2, C, H), jnp.float32),
            pltpu.VMEM((2, C, H), jnp.float32),
            pltpu.VMEM((2, C), jnp.float32),
            pltpu.VMEM((16,), jnp.float32),
            pltpu.VMEM((H,), jnp.float32),
            pltpu.SemaphoreType.DMA,
            pltpu.SemaphoreType.DMA,
            pltpu.SemaphoreType.DMA,
            pltpu.SemaphoreType.DMA,
            pltpu.SemaphoreType.DMA,
        ],
    )


def _final_body(src_h, dst_h, g1_h, g2_h, dv_h, out_o,
                sidx, didx, g1b, g2b, dvb, ob, sem):
    cid = lax.axis_index("c")
    sid = lax.axis_index("s")
    wid = cid * NS + sid

    def chunk(ci, carry):
        base = wid * EPW + ci * C
        pltpu.sync_copy(src_h.at[pl.ds(base, C)], sidx)
        pltpu.sync_copy(dst_h.at[pl.ds(base, C)], didx)
        cp_a = pltpu.async_copy(g1_h.at[sidx], g1b, sem)
        cp_b = pltpu.async_copy(g2_h.at[didx], g2b, sem)
        cp_d = pltpu.async_copy(dv_h.at[pl.ds(base, C)], dvb, sem)
        cp_a.wait()
        cp_b.wait()
        cp_d.wait()

        def group(g, c2):
            sl = pl.ds(g * 16, 16)
            z = g1b[sl] + g2b[sl] + dvb[sl]
            ob[sl] = 1.0 / (1.0 + jnp.exp(-z))
            return c2

        lax.fori_loop(0, C // 16, group, 0, unroll=False)
        pltpu.sync_copy(ob, out_o.at[pl.ds(base, C)])
        return carry

    lax.fori_loop(0, NCHUNK, chunk, 0, unroll=False)


@functools.cache
def _final_pass():
    return pl.kernel(
        _final_body,
        out_type=jax.ShapeDtypeStruct((EP,), jnp.float32),
        mesh=_mesh(),
        compiler_params=_cparams(),
        scratch_types=[
            pltpu.VMEM((C,), jnp.int32),
            pltpu.VMEM((C,), jnp.int32),
            pltpu.VMEM((C,), jnp.float32),
            pltpu.VMEM((C,), jnp.float32),
            pltpu.VMEM((C,), jnp.float32),
            pltpu.VMEM((C,), jnp.float32),
            pltpu.SemaphoreType.DMA,
        ],
    )


def _pad_nodes(x, width):
    out = jnp.zeros((NP, width), jnp.float32)
    return out.at[:N].set(x)


def _conv(h, src, dst, eaW, AW, p):
    q = h @ p["Wq"] + p["bq"]
    k = h @ p["Wk"] + p["bk"]
    v = h @ p["Wv"] + p["bv"]
    kv = _pad_nodes(jnp.concatenate([k, v], axis=1), 2 * H)
    K = jnp.max(jnp.linalg.norm(k, axis=1))
    s = jnp.linalg.norm(q, axis=1) * (K + AW) * RSQRT_H
    qpad = _pad_nodes(
        jnp.concatenate([q * RSQRT_H, -s[:, None],
                         jnp.zeros((N, HQ - H - 1), jnp.float32)], axis=1), HQ)
    z128 = jnp.zeros((NP, H), jnp.float32)
    z1 = jnp.zeros((NP,), jnp.float32)
    acc, den = _conv_edge_pass()(src, dst, qpad, kv, eaW, z128, z1)
    num = acc[0, :N] + acc[1, :N]
    denom = den[0, :N] + den[1, :N]
    out = jnp.where(denom[:, None] > 0, num / denom[:, None], 0.0)
    return out + h @ p["Ws"] + p["bs"]


def kernel(x, edge_attr, edge_index, params):
    p = params
    src = jnp.concatenate([edge_index[0], jnp.zeros((EP - E,), jnp.int32)])
    dst = jnp.concatenate([edge_index[1],
                           jnp.full((EP - E,), DPAD, jnp.int32)])
    eap = jnp.zeros((EP, 16), jnp.float32).at[:E].set(edge_attr)
    h = x @ p["node_W"] + p["node_b"]

    c1 = p["conv1"]
    eaW1 = eap @ (p["edge_W"] @ c1["We"]) + p["edge_b"] @ c1["We"]
    AW1 = jnp.max(jnp.linalg.norm(eaW1[:E], axis=1))
    h = jax.nn.leaky_relu(_conv(h, src, dst, eaW1, AW1, c1))

    w = p["eu3_W"] @ p["fc_W"]
    c = p["eu3_b"] @ p["fc_W"] + p["fc_b"]

    A1 = _pad_nodes(h @ p["eu1_W"][:H], H)
    B1 = _pad_nodes(h @ p["eu1_W"][H:2 * H], H)
    T1 = eap @ (p["edge_W"] @ p["eu1_W"][2 * H:]) + (
        p["edge_b"] @ p["eu1_W"][2 * H:] + p["eu1_b"])
    ea2, rm1, _ = _eu_pass()(src, dst, A1, B1, T1, jnp.zeros((H,), jnp.float32))
    Aea2 = jnp.sqrt(jnp.max(rm1))

    c2 = p["conv2"]
    eaW2 = ea2 @ c2["We"]
    AW2 = Aea2 * jnp.linalg.norm(c2["We"])
    h = jax.nn.leaky_relu(_conv(h, src, dst, eaW2, AW2, c2))

    A2 = _pad_nodes(h @ p["eu2_W"][:H], H)
    B2 = _pad_nodes(h @ p["eu2_W"][H:2 * H], H)
    T2 = ea2 @ p["eu2_W"][2 * H:] + p["eu2_b"]
    ea3, rm2, dvec = _eu_pass()(src, dst, A2, B2, T2, w[2 * H:, 0])
    Aea3 = jnp.sqrt(jnp.max(rm2))

    c3 = p["conv3"]
    eaW3 = ea3 @ c3["We"]
    AW3 = Aea3 * jnp.linalg.norm(c3["We"])
    h = _conv(h, src, dst, eaW3, AW3, c3)

    g1 = jnp.zeros((NP,), jnp.float32).at[:N].set((h @ w[:H])[:, 0])
    g2 = jnp.zeros((NP,), jnp.float32).at[:N].set((h @ w[H:2 * H])[:, 0] + c[0])
    out = _final_pass()(src, dst, g1, g2, dvec)
    return out[:E, None]


# speedup vs baseline: 1.1356x; 1.0266x over previous
import functools
import numpy as np
import jax
import jax.numpy as jnp
from jax import lax
from jax.experimental import pallas as pl
from jax.experimental.pallas import tpu as pltpu
from jax.experimental.pallas import tpu_sc as plsc

N = 10000
E = 320000
H = 128
RSQRT_H = float(1.0 / np.sqrt(128.0))

NC = 2
NS = 16
NW = NC * NS
C = 64
EP = 321536
EPW = EP // NW
NCHUNK = EPW // C
NP = 10112
NPT = NP // NS
HQ = 144
DPAD = 10104

@functools.cache
def _mesh():
    return plsc.VectorSubcoreMesh(core_axis_name="c", subcore_axis_name="s",
                                  num_cores=NC, num_subcores=NS)

_CPARAMS = None


def _cparams():
    return pltpu.CompilerParams(use_tc_tiling_on_sc=False,
                                needs_layout_passes=False)


CV = 80
EPW_CV = E // NW
NCHUNK_CV = EPW_CV // CV


def _conv_edge_body(src_h, dst_h, q_h, kv_h, ew_h, z128_h, z1_h,
                    acc_o, den_o,
                    sidx2, didx2, qb, kvb, ewb, exb,
                    acc_sp, den_sp,
                    sem_q, sem_kv, sem_ew, sem_sc, sem_ix):
    cid = lax.axis_index("c")
    sid = lax.axis_index("s")
    wid = cid * NS + sid
    row0 = sid * NPT
    pltpu.sync_copy(z128_h.at[pl.ds(row0, NPT)], acc_sp.at[pl.ds(row0, NPT)])
    pltpu.sync_copy(z1_h.at[pl.ds(row0, NPT)], den_sp.at[pl.ds(row0, NPT)])
    plsc.subcore_barrier()

    e0 = wid * EPW_CV
    pltpu.sync_copy(src_h.at[pl.ds(e0, CV)], sidx2.at[0])
    pltpu.sync_copy(dst_h.at[pl.ds(e0, CV)], didx2.at[0])
    pltpu.async_copy(q_h.at[didx2.at[0]], qb, sem_q)
    pltpu.async_copy(kv_h.at[sidx2.at[0]], kvb, sem_kv)
    pltpu.async_copy(ew_h.at[pl.ds(e0, CV)], ewb, sem_ew)
    pltpu.async_copy(src_h.at[pl.ds(e0 + CV, CV)], sidx2.at[1], sem_ix)
    pltpu.async_copy(dst_h.at[pl.ds(e0 + CV, CV)], didx2.at[1], sem_ix)

    lanes = lax.iota(jnp.int32, 16)
    perms = [lanes ^ sh for sh in (8, 4, 2, 1)]

    def compute():
        def group(g, c2):
            av = jnp.zeros((16,), jnp.float32)
            for j in range(16):
                e = g * 16 + j
                acc = jnp.zeros((16,), jnp.float32)
                for t in range(8):
                    sl = pl.ds(t * 16, 16)
                    acc = acc + qb[e, sl] * (kvb[e, sl] + ewb[e, sl])
                acc = acc + qb[e, pl.ds(128, 16)]
                for perm in perms:
                    acc = acc + acc.at[perm].get(mode="promise_in_bounds")
                av = jnp.where(lanes == j, acc, av)
            exv = jnp.exp(av)
            exb[pl.ds(g * 16, 16)] = exv
            for j in range(16):
                e = g * 16 + j
                x = exv[j]
                for t in range(8):
                    ewb[e, pl.ds(t * 16, 16)] = x * (
                        kvb[e, pl.ds(128 + t * 16, 16)] + ewb[e, pl.ds(t * 16, 16)])
            return c2
        lax.fori_loop(0, CV // 16, group, 0, unroll=False)

    def iter_body(ci, carry):
        cur = lax.rem(ci, 2)
        nxt = 1 - cur
        base = e0 + ci * CV
        pltpu.make_async_copy(q_h.at[didx2.at[cur]], qb, sem_q).wait()
        pltpu.make_async_copy(kv_h.at[sidx2.at[cur]], kvb, sem_kv).wait()
        pltpu.make_async_copy(ew_h.at[pl.ds(base, CV)], ewb, sem_ew).wait()
        compute()
        cp_acc = pltpu.async_copy(ewb, acc_sp.at[didx2.at[cur]], sem_sc, add=True)
        cp_den = pltpu.async_copy(exb, den_sp.at[didx2.at[cur]], sem_sc, add=True)
        nbase = e0 + (ci + 1) * CV

        @pl.when(ci + 1 < NCHUNK_CV)
        def _():
            pltpu.make_async_copy(src_h.at[pl.ds(nbase, CV)], sidx2.at[nxt], sem_ix).wait()
            pltpu.make_async_copy(dst_h.at[pl.ds(nbase, CV)], didx2.at[nxt], sem_ix).wait()
            pltpu.async_copy(q_h.at[didx2.at[nxt]], qb, sem_q)
            pltpu.async_copy(kv_h.at[sidx2.at[nxt]], kvb, sem_kv)

        cp_acc.wait()
        cp_den.wait()

        @pl.when(ci + 1 < NCHUNK_CV)
        def _():
            pltpu.async_copy(ew_h.at[pl.ds(nbase, CV)], ewb, sem_ew)

        @pl.when(ci + 2 < NCHUNK_CV)
        def _():
            pbase = e0 + (ci + 2) * CV
            pltpu.async_copy(src_h.at[pl.ds(pbase, CV)], sidx2.at[cur], sem_ix)
            pltpu.async_copy(dst_h.at[pl.ds(pbase, CV)], didx2.at[cur], sem_ix)

        return carry

    lax.fori_loop(0, NCHUNK_CV, iter_body, 0, unroll=False)
    plsc.subcore_barrier()
    pltpu.sync_copy(acc_sp.at[pl.ds(row0, NPT)], acc_o.at[cid, pl.ds(row0, NPT)])
    pltpu.sync_copy(den_sp.at[pl.ds(row0, NPT)], den_o.at[cid, pl.ds(row0, NPT)])


@functools.cache
def _conv_edge_pass():
    return pl.kernel(
        _conv_edge_body,
        out_type=[jax.ShapeDtypeStruct((NC, NP, H), jnp.float32),
                  jax.ShapeDtypeStruct((NC, NP), jnp.float32)],
        mesh=_mesh(),
        compiler_params=_cparams(),
        scratch_types=[
            pltpu.VMEM((2, CV), jnp.int32),
            pltpu.VMEM((2, CV), jnp.int32),
            pltpu.VMEM((CV, HQ), jnp.float32),
            pltpu.VMEM((CV, 2 * H), jnp.float32),
            pltpu.VMEM((CV, H), jnp.float32),
            pltpu.VMEM((CV,), jnp.float32),
            pltpu.VMEM_SHARED((NP, H), jnp.float32),
            pltpu.VMEM_SHARED((NP,), jnp.float32),
            pltpu.SemaphoreType.DMA,
            pltpu.SemaphoreType.DMA,
            pltpu.SemaphoreType.DMA,
            pltpu.SemaphoreType.DMA,
            pltpu.SemaphoreType.DMA,
        ],
    )


def _eu_body(src_h, dst_h, a_h, b_h, t_h, wv_h,
             ea_o, rm_o, dv_o,
             sidx3, didx3, ab2, bb2, tb2, dvb2, rmb, wvb,
             sem_a, sem_b, sem_t, sem_st, sem_ix):
    cid = lax.axis_index("c")
    sid = lax.axis_index("s")
    wid = cid * NS + sid
    pltpu.sync_copy(wv_h, wvb)
    e0 = wid * EPW
    pltpu.sync_copy(src_h.at[pl.ds(e0, C)], sidx3.at[0])
    pltpu.sync_copy(dst_h.at[pl.ds(e0, C)], didx3.at[0])
    pltpu.async_copy(a_h.at[sidx3.at[0]], ab2.at[0], sem_a)
    pltpu.async_copy(b_h.at[didx3.at[0]], bb2.at[0], sem_b)
    pltpu.async_copy(t_h.at[pl.ds(e0, C)], tb2.at[0], sem_t)
    pltpu.async_copy(src_h.at[pl.ds(e0 + C, C)], sidx3.at[1], sem_ix)
    pltpu.async_copy(dst_h.at[pl.ds(e0 + C, C)], didx3.at[1], sem_ix)

    lanes = lax.iota(jnp.int32, 16)
    perms = [lanes ^ sh for sh in (8, 4, 2, 1)]

    def st_waits(par, base):
        pltpu.make_async_copy(tb2.at[par], ea_o.at[pl.ds(base, C)], sem_st).wait()
        pltpu.make_async_copy(dvb2.at[par], dv_o.at[pl.ds(base, C)], sem_st).wait()

    def iter_body(ci, rm):
        cur = lax.rem(ci, 2)
        nxt = 1 - cur
        ix = lax.rem(ci, 3)
        base = e0 + ci * C
        pltpu.make_async_copy(a_h.at[sidx3.at[ix]], ab2.at[cur], sem_a).wait()
        pltpu.make_async_copy(b_h.at[didx3.at[ix]], bb2.at[cur], sem_b).wait()
        pltpu.make_async_copy(t_h.at[pl.ds(base, C)], tb2.at[cur], sem_t).wait()

        def group(g, rm2):
            dvv = jnp.zeros((16,), jnp.float32)
            for j in range(16):
                e = g * 16 + j
                sq = jnp.zeros((16,), jnp.float32)
                dv = jnp.zeros((16,), jnp.float32)
                for t in range(8):
                    sl = pl.ds(t * 16, 16)
                    z = ab2[cur, e, sl] + bb2[cur, e, sl] + tb2[cur, e, sl]
                    z = jnp.where(z > 0, z, z * jnp.float32(0.01))
                    tb2[cur, e, sl] = z
                    sq = sq + z * z
                    dv = dv + z * wvb[sl]
                for perm in perms:
                    sq = sq + sq.at[perm].get(mode="promise_in_bounds")
                    dv = dv + dv.at[perm].get(mode="promise_in_bounds")
                rm2 = jnp.maximum(rm2, sq)
                dvv = jnp.where(lanes == j, dv, dvv)
            dvb2[cur, pl.ds(g * 16, 16)] = dvv
            return rm2

        rm = lax.fori_loop(0, C // 16, group, rm, unroll=False)
        pltpu.async_copy(tb2.at[cur], ea_o.at[pl.ds(base, C)], sem_st)
        pltpu.async_copy(dvb2.at[cur], dv_o.at[pl.ds(base, C)], sem_st)

        @pl.when(ci + 1 < NCHUNK)
        def _():
            nix = lax.rem(ci + 1, 3)
            nbase = base + C
            pltpu.make_async_copy(src_h.at[pl.ds(nbase, C)], sidx3.at[nix], sem_ix).wait()
            pltpu.make_async_copy(dst_h.at[pl.ds(nbase, C)], didx3.at[nix], sem_ix).wait()
            pltpu.async_copy(a_h.at[sidx3.at[nix]], ab2.at[nxt], sem_a)
            pltpu.async_copy(b_h.at[didx3.at[nix]], bb2.at[nxt], sem_b)

            @pl.when(ci > 0)
            def _():
                st_waits(nxt, base - C)

            pltpu.async_copy(t_h.at[pl.ds(nbase, C)], tb2.at[nxt], sem_t)

            @pl.when(ci + 2 < NCHUNK)
            def _():
                pix = lax.rem(ci + 2, 3)
                pbase = base + 2 * C
                pltpu.async_copy(src_h.at[pl.ds(pbase, C)], sidx3.at[pix], sem_ix)
                pltpu.async_copy(dst_h.at[pl.ds(pbase, C)], didx3.at[pix], sem_ix)

        return rm

    rm = lax.fori_loop(0, NCHUNK, iter_body, jnp.zeros((16,), jnp.float32),
                       unroll=False)
    st_waits(0, e0)
    st_waits(1, e0)
    rmb[...] = rm
    pltpu.sync_copy(rmb, rm_o.at[cid, sid])


@functools.cache
def _eu_pass():
    return pl.kernel(
        _eu_body,
        out_type=[jax.ShapeDtypeStruct((EP, H), jnp.float32),
                  jax.ShapeDtypeStruct((NC, NS, 16), jnp.float32),
                  jax.ShapeDtypeStruct((EP,), jnp.float32)],
        mesh=_mesh(),
        compiler_params=_cparams(),
        scratch_types=[
            pltpu.VMEM((3, C), jnp.int32),
            pltpu.VMEM((3, C), jnp.int32),
            pltpu.VMEM((2, C, H), jnp.float32),
            pltpu.VMEM((2, C, H), jnp.float32),
            pltpu.VMEM((2, C, H), jnp.float32),
            pltpu.VMEM((2, C), jnp.float32),
            pltpu.VMEM((16,), jnp.float32),
            pltpu.VMEM((H,), jnp.float32),
            pltpu.SemaphoreType.DMA,
            pltpu.SemaphoreType.DMA,
            pltpu.SemaphoreType.DMA,
            pltpu.SemaphoreType.DMA,
            pltpu.SemaphoreType.DMA,
        ],
    )


def _final_body(src_h, dst_h, g1_h, g2_h, dv_h, out_o,
                sidx3, didx3, g1b2, g2b2, dvb2, ob2,
                sem_a, sem_b, sem_d, sem_st, sem_ix):
    cid = lax.axis_index("c")
    sid = lax.axis_index("s")
    wid = cid * NS + sid
    e0 = wid * EPW
    pltpu.sync_copy(src_h.at[pl.ds(e0, C)], sidx3.at[0])
    pltpu.sync_copy(dst_h.at[pl.ds(e0, C)], didx3.at[0])
    pltpu.async_copy(g1_h.at[sidx3.at[0]], g1b2.at[0], sem_a)
    pltpu.async_copy(g2_h.at[didx3.at[0]], g2b2.at[0], sem_b)
    pltpu.async_copy(dv_h.at[pl.ds(e0, C)], dvb2.at[0], sem_d)
    pltpu.async_copy(src_h.at[pl.ds(e0 + C, C)], sidx3.at[1], sem_ix)
    pltpu.async_copy(dst_h.at[pl.ds(e0 + C, C)], didx3.at[1], sem_ix)

    def iter_body(ci, carry):
        cur = lax.rem(ci, 2)
        nxt = 1 - cur
        ix = lax.rem(ci, 3)
        base = e0 + ci * C
        pltpu.make_async_copy(g1_h.at[sidx3.at[ix]], g1b2.at[cur], sem_a).wait()
        pltpu.make_async_copy(g2_h.at[didx3.at[ix]], g2b2.at[cur], sem_b).wait()
        pltpu.make_async_copy(dv_h.at[pl.ds(base, C)], dvb2.at[cur], sem_d).wait()

        def group(g, c2):
            sl = pl.ds(g * 16, 16)
            z = g1b2[cur, sl] + g2b2[cur, sl] + dvb2[cur, sl]
            ob2[cur, sl] = 1.0 / (1.0 + jnp.exp(-z))
            return c2

        lax.fori_loop(0, C // 16, group, 0, unroll=False)
        pltpu.async_copy(ob2.at[cur], out_o.at[pl.ds(base, C)], sem_st)

        @pl.when(ci + 1 < NCHUNK)
        def _():
            nix = lax.rem(ci + 1, 3)
            nbase = base + C
            pltpu.make_async_copy(src_h.at[pl.ds(nbase, C)], sidx3.at[nix], sem_ix).wait()
            pltpu.make_async_copy(dst_h.at[pl.ds(nbase, C)], didx3.at[nix], sem_ix).wait()
            pltpu.async_copy(g1_h.at[sidx3.at[nix]], g1b2.at[nxt], sem_a)
            pltpu.async_copy(g2_h.at[didx3.at[nix]], g2b2.at[nxt], sem_b)

            @pl.when(ci > 0)
            def _():
                pltpu.make_async_copy(ob2.at[nxt], out_o.at[pl.ds(base - C, C)],
                                      sem_st).wait()

            pltpu.async_copy(dv_h.at[pl.ds(nbase, C)], dvb2.at[nxt], sem_d)

            @pl.when(ci + 2 < NCHUNK)
            def _():
                pix = lax.rem(ci + 2, 3)
                pbase = base + 2 * C
                pltpu.async_copy(src_h.at[pl.ds(pbase, C)], sidx3.at[pix], sem_ix)
                pltpu.async_copy(dst_h.at[pl.ds(pbase, C)], didx3.at[pix], sem_ix)

        return carry

    lax.fori_loop(0, NCHUNK, iter_body, 0, unroll=False)
    pltpu.make_async_copy(ob2.at[0], out_o.at[pl.ds(e0, C)], sem_st).wait()
    pltpu.make_async_copy(ob2.at[1], out_o.at[pl.ds(e0, C)], sem_st).wait()


@functools.cache
def _final_pass():
    return pl.kernel(
        _final_body,
        out_type=jax.ShapeDtypeStruct((EP,), jnp.float32),
        mesh=_mesh(),
        compiler_params=_cparams(),
        scratch_types=[
            pltpu.VMEM((3, C), jnp.int32),
            pltpu.VMEM((3, C), jnp.int32),
            pltpu.VMEM((2, C), jnp.float32),
            pltpu.VMEM((2, C), jnp.float32),
            pltpu.VMEM((2, C), jnp.float32),
            pltpu.VMEM((2, C), jnp.float32),
            pltpu.SemaphoreType.DMA,
            pltpu.SemaphoreType.DMA,
            pltpu.SemaphoreType.DMA,
            pltpu.SemaphoreType.DMA,
            pltpu.SemaphoreType.DMA,
        ],
    )


def _pad_nodes(x, width):
    out = jnp.zeros((NP, width), jnp.float32)
    return out.at[:N].set(x)


def _conv(h, src, dst, eaW, AW, p):
    q = h @ p["Wq"] + p["bq"]
    k = h @ p["Wk"] + p["bk"]
    v = h @ p["Wv"] + p["bv"]
    kv = _pad_nodes(jnp.concatenate([k, v], axis=1), 2 * H)
    K = jnp.max(jnp.linalg.norm(k, axis=1))
    s = jnp.linalg.norm(q, axis=1) * (K + AW) * RSQRT_H
    qpad = _pad_nodes(
        jnp.concatenate([q * RSQRT_H, -s[:, None],
                         jnp.zeros((N, HQ - H - 1), jnp.float32)], axis=1), HQ)
    z128 = jnp.zeros((NP, H), jnp.float32)
    z1 = jnp.zeros((NP,), jnp.float32)
    acc, den = _conv_edge_pass()(src, dst, qpad, kv, eaW, z128, z1)
    num = acc[0, :N] + acc[1, :N]
    denom = den[0, :N] + den[1, :N]
    out = jnp.where(denom[:, None] > 0, num / denom[:, None], 0.0)
    return out + h @ p["Ws"] + p["bs"]


def kernel(x, edge_attr, edge_index, params):
    p = params
    src = jnp.concatenate([edge_index[0], jnp.zeros((EP - E,), jnp.int32)])
    dst = jnp.concatenate([edge_index[1],
                           jnp.full((EP - E,), DPAD, jnp.int32)])
    eap = jnp.zeros((EP, 16), jnp.float32).at[:E].set(edge_attr)
    h = x @ p["node_W"] + p["node_b"]

    c1 = p["conv1"]
    eaW1 = eap @ (p["edge_W"] @ c1["We"]) + p["edge_b"] @ c1["We"]
    AW1 = jnp.max(jnp.linalg.norm(eaW1[:E], axis=1))
    h = jax.nn.leaky_relu(_conv(h, src, dst, eaW1, AW1, c1))

    w = p["eu3_W"] @ p["fc_W"]
    c = p["eu3_b"] @ p["fc_W"] + p["fc_b"]

    A1 = _pad_nodes(h @ p["eu1_W"][:H], H)
    B1 = _pad_nodes(h @ p["eu1_W"][H:2 * H], H)
    T1 = eap @ (p["edge_W"] @ p["eu1_W"][2 * H:]) + (
        p["edge_b"] @ p["eu1_W"][2 * H:] + p["eu1_b"])
    ea2, rm1, _ = _eu_pass()(src, dst, A1, B1, T1, jnp.zeros((H,), jnp.float32))
    Aea2 = jnp.sqrt(jnp.max(rm1))

    c2 = p["conv2"]
    eaW2 = ea2 @ c2["We"]
    AW2 = Aea2 * jnp.linalg.norm(c2["We"])
    h = jax.nn.leaky_relu(_conv(h, src, dst, eaW2, AW2, c2))

    A2 = _pad_nodes(h @ p["eu2_W"][:H], H)
    B2 = _pad_nodes(h @ p["eu2_W"][H:2 * H], H)
    T2 = ea2 @ p["eu2_W"][2 * H:] + p["eu2_b"]
    ea3, rm2, dvec = _eu_pass()(src, dst, A2, B2, T2, w[2 * H:, 0])
    Aea3 = jnp.sqrt(jnp.max(rm2))

    c3 = p["conv3"]
    eaW3 = ea3 @ c3["We"]
    AW3 = Aea3 * jnp.linalg.norm(c3["We"])
    h = _conv(h, src, dst, eaW3, AW3, c3)

    g1 = jnp.zeros((NP,), jnp.float32).at[:N].set((h @ w[:H])[:, 0])
    g2 = jnp.zeros((NP,), jnp.float32).at[:N].set((h @ w[H:2 * H])[:, 0] + c[0])
    out = _final_pass()(src, dst, g1, g2, dvec)
    return out[:E, None]


# speedup vs baseline: 1.1364x; 1.0007x over previous
import functools
import numpy as np
import jax
import jax.numpy as jnp
from jax import lax
from jax.experimental import pallas as pl
from jax.experimental.pallas import tpu as pltpu
from jax.experimental.pallas import tpu_sc as plsc

N = 10000
E = 320000
H = 128
RSQRT_H = float(1.0 / np.sqrt(128.0))

NC = 2
NS = 16
NW = NC * NS
C = 64
EP = 321536
EPW = EP // NW
NCHUNK = EPW // C
NP = 10112
NPT = NP // NS
HQ = 144
DPAD = 10104

@functools.cache
def _mesh():
    return plsc.VectorSubcoreMesh(core_axis_name="c", subcore_axis_name="s",
                                  num_cores=NC, num_subcores=NS)

def _cparams():
    return pltpu.CompilerParams(use_tc_tiling_on_sc=False,
                                needs_layout_passes=False)


CV = 80
EPW_CV = E // NW
NCHUNK_CV = EPW_CV // CV


def _conv_edge_body(src_h, dst_h, q_h, kv_h, ew_h, z128_h, z1_h,
                    acc_o, den_o,
                    sidx2, didx2, qb, kvb, ewb, exb,
                    acc_sp, den_sp,
                    sem_q, sem_kv, sem_ew, sem_sc, sem_ix):
    cid = lax.axis_index("c")
    sid = lax.axis_index("s")
    wid = cid * NS + sid
    row0 = sid * NPT
    pltpu.sync_copy(z128_h.at[pl.ds(row0, NPT)], acc_sp.at[pl.ds(row0, NPT)])
    pltpu.sync_copy(z1_h.at[pl.ds(row0, NPT)], den_sp.at[pl.ds(row0, NPT)])
    plsc.subcore_barrier()

    e0 = wid * EPW_CV
    pltpu.sync_copy(src_h.at[pl.ds(e0, CV)], sidx2.at[0])
    pltpu.sync_copy(dst_h.at[pl.ds(e0, CV)], didx2.at[0])
    pltpu.async_copy(q_h.at[didx2.at[0]], qb, sem_q)
    pltpu.async_copy(kv_h.at[sidx2.at[0]], kvb, sem_kv)
    pltpu.async_copy(ew_h.at[pl.ds(e0, CV)], ewb, sem_ew)
    pltpu.async_copy(src_h.at[pl.ds(e0 + CV, CV)], sidx2.at[1], sem_ix)
    pltpu.async_copy(dst_h.at[pl.ds(e0 + CV, CV)], didx2.at[1], sem_ix)

    lanes = lax.iota(jnp.int32, 16)
    perms = [lanes ^ sh for sh in (8, 4, 2, 1)]

    def compute():
        def group(g, c2):
            av = jnp.zeros((16,), jnp.float32)
            for j in range(16):
                e = g * 16 + j
                acc = jnp.zeros((16,), jnp.float32)
                for t in range(8):
                    sl = pl.ds(t * 16, 16)
                    acc = acc + qb[e, sl] * (kvb[e, sl] + ewb[e, sl])
                acc = acc + qb[e, pl.ds(128, 16)]
                for perm in perms:
                    acc = acc + acc.at[perm].get(mode="promise_in_bounds")
                av = jnp.where(lanes == j, acc, av)
            exv = jnp.exp(av)
            exb[pl.ds(g * 16, 16)] = exv
            for j in range(16):
                e = g * 16 + j
                x = exv[j]
                for t in range(8):
                    ewb[e, pl.ds(t * 16, 16)] = x * (
                        kvb[e, pl.ds(128 + t * 16, 16)] + ewb[e, pl.ds(t * 16, 16)])
            return c2
        lax.fori_loop(0, CV // 16, group, 0, unroll=False)

    def iter_body(ci, carry):
        cur = lax.rem(ci, 2)
        nxt = 1 - cur
        base = e0 + ci * CV
        pltpu.make_async_copy(q_h.at[didx2.at[cur]], qb, sem_q).wait()
        pltpu.make_async_copy(kv_h.at[sidx2.at[cur]], kvb, sem_kv).wait()
        pltpu.make_async_copy(ew_h.at[pl.ds(base, CV)], ewb, sem_ew).wait()
        compute()
        cp_acc = pltpu.async_copy(ewb, acc_sp.at[didx2.at[cur]], sem_sc, add=True)
        cp_den = pltpu.async_copy(exb, den_sp.at[didx2.at[cur]], sem_sc, add=True)
        nbase = e0 + (ci + 1) * CV

        @pl.when(ci + 1 < NCHUNK_CV)
        def _():
            pltpu.make_async_copy(src_h.at[pl.ds(nbase, CV)], sidx2.at[nxt], sem_ix).wait()
            pltpu.make_async_copy(dst_h.at[pl.ds(nbase, CV)], didx2.at[nxt], sem_ix).wait()
            pltpu.async_copy(q_h.at[didx2.at[nxt]], qb, sem_q)
            pltpu.async_copy(kv_h.at[sidx2.at[nxt]], kvb, sem_kv)

        cp_acc.wait()
        cp_den.wait()

        @pl.when(ci + 1 < NCHUNK_CV)
        def _():
            pltpu.async_copy(ew_h.at[pl.ds(nbase, CV)], ewb, sem_ew)

        @pl.when(ci + 2 < NCHUNK_CV)
        def _():
            pbase = e0 + (ci + 2) * CV
            pltpu.async_copy(src_h.at[pl.ds(pbase, CV)], sidx2.at[cur], sem_ix)
            pltpu.async_copy(dst_h.at[pl.ds(pbase, CV)], didx2.at[cur], sem_ix)

        return carry

    lax.fori_loop(0, NCHUNK_CV, iter_body, 0, unroll=False)
    plsc.subcore_barrier()
    pltpu.sync_copy(acc_sp.at[pl.ds(row0, NPT)], acc_o.at[cid, pl.ds(row0, NPT)])
    pltpu.sync_copy(den_sp.at[pl.ds(row0, NPT)], den_o.at[cid, pl.ds(row0, NPT)])


@functools.cache
def _conv_edge_pass():
    return pl.kernel(
        _conv_edge_body,
        out_type=[jax.ShapeDtypeStruct((NC, NP, H), jnp.float32),
                  jax.ShapeDtypeStruct((NC, NP), jnp.float32)],
        mesh=_mesh(),
        compiler_params=_cparams(),
        scratch_types=[
            pltpu.VMEM((2, CV), jnp.int32),
            pltpu.VMEM((2, CV), jnp.int32),
            pltpu.VMEM((CV, HQ), jnp.float32),
            pltpu.VMEM((CV, 2 * H), jnp.float32),
            pltpu.VMEM((CV, H), jnp.float32),
            pltpu.VMEM((CV,), jnp.float32),
            pltpu.VMEM_SHARED((NP, H), jnp.float32),
            pltpu.VMEM_SHARED((NP,), jnp.float32),
            pltpu.SemaphoreType.DMA,
            pltpu.SemaphoreType.DMA,
            pltpu.SemaphoreType.DMA,
            pltpu.SemaphoreType.DMA,
            pltpu.SemaphoreType.DMA,
        ],
    )


def _eu_body(src_h, dst_h, a_h, b_h, t_h, wv_h,
             ea_o, rm_o, dv_o,
             sidx3, didx3, ab2, bb2, tb2, dvb2, rmb, wvb,
             sem_a, sem_b, sem_t, sem_st, sem_ix):
    cid = lax.axis_index("c")
    sid = lax.axis_index("s")
    wid = cid * NS + sid
    pltpu.sync_copy(wv_h, wvb)
    e0 = wid * EPW
    pltpu.sync_copy(src_h.at[pl.ds(e0, C)], sidx3.at[0])
    pltpu.sync_copy(dst_h.at[pl.ds(e0, C)], didx3.at[0])
    pltpu.async_copy(a_h.at[sidx3.at[0]], ab2.at[0], sem_a)
    pltpu.async_copy(b_h.at[didx3.at[0]], bb2.at[0], sem_b)
    pltpu.async_copy(t_h.at[pl.ds(e0, C)], tb2.at[0], sem_t)
    pltpu.async_copy(src_h.at[pl.ds(e0 + C, C)], sidx3.at[1], sem_ix)
    pltpu.async_copy(dst_h.at[pl.ds(e0 + C, C)], didx3.at[1], sem_ix)

    lanes = lax.iota(jnp.int32, 16)
    perms = [lanes ^ sh for sh in (8, 4, 2, 1)]

    def st_waits(par, base):
        pltpu.make_async_copy(tb2.at[par], ea_o.at[pl.ds(base, C)], sem_st).wait()
        pltpu.make_async_copy(dvb2.at[par], dv_o.at[pl.ds(base, C)], sem_st).wait()

    def iter_body(ci, rm):
        cur = lax.rem(ci, 2)
        nxt = 1 - cur
        ix = lax.rem(ci, 3)
        base = e0 + ci * C
        pltpu.make_async_copy(a_h.at[sidx3.at[ix]], ab2.at[cur], sem_a).wait()
        pltpu.make_async_copy(b_h.at[didx3.at[ix]], bb2.at[cur], sem_b).wait()
        pltpu.make_async_copy(t_h.at[pl.ds(base, C)], tb2.at[cur], sem_t).wait()

        def group(g, rm2):
            dvv = jnp.zeros((16,), jnp.float32)
            for j in range(16):
                e = g * 16 + j
                sq = jnp.zeros((16,), jnp.float32)
                dv = jnp.zeros((16,), jnp.float32)
                for t in range(8):
                    sl = pl.ds(t * 16, 16)
                    z = ab2[cur, e, sl] + bb2[cur, e, sl] + tb2[cur, e, sl]
                    z = jnp.where(z > 0, z, z * jnp.float32(0.01))
                    tb2[cur, e, sl] = z
                    sq = sq + z * z
                    dv = dv + z * wvb[sl]
                for perm in perms:
                    sq = sq + sq.at[perm].get(mode="promise_in_bounds")
                    dv = dv + dv.at[perm].get(mode="promise_in_bounds")
                rm2 = jnp.maximum(rm2, sq)
                dvv = jnp.where(lanes == j, dv, dvv)
            dvb2[cur, pl.ds(g * 16, 16)] = dvv
            return rm2

        rm = lax.fori_loop(0, C // 16, group, rm, unroll=False)
        pltpu.async_copy(tb2.at[cur], ea_o.at[pl.ds(base, C)], sem_st)
        pltpu.async_copy(dvb2.at[cur], dv_o.at[pl.ds(base, C)], sem_st)

        @pl.when(ci + 1 < NCHUNK)
        def _():
            nix = lax.rem(ci + 1, 3)
            nbase = base + C
            pltpu.make_async_copy(src_h.at[pl.ds(nbase, C)], sidx3.at[nix], sem_ix).wait()
            pltpu.make_async_copy(dst_h.at[pl.ds(nbase, C)], didx3.at[nix], sem_ix).wait()
            pltpu.async_copy(a_h.at[sidx3.at[nix]], ab2.at[nxt], sem_a)
            pltpu.async_copy(b_h.at[didx3.at[nix]], bb2.at[nxt], sem_b)

            @pl.when(ci > 0)
            def _():
                st_waits(nxt, base - C)

            pltpu.async_copy(t_h.at[pl.ds(nbase, C)], tb2.at[nxt], sem_t)

            @pl.when(ci + 2 < NCHUNK)
            def _():
                pix = lax.rem(ci + 2, 3)
                pbase = base + 2 * C
                pltpu.async_copy(src_h.at[pl.ds(pbase, C)], sidx3.at[pix], sem_ix)
                pltpu.async_copy(dst_h.at[pl.ds(pbase, C)], didx3.at[pix], sem_ix)

        return rm

    rm = lax.fori_loop(0, NCHUNK, iter_body, jnp.zeros((16,), jnp.float32),
                       unroll=False)
    st_waits(0, e0)
    st_waits(1, e0)
    rmb[...] = rm
    pltpu.sync_copy(rmb, rm_o.at[cid, sid])


@functools.cache
def _eu_pass():
    return pl.kernel(
        _eu_body,
        out_type=[jax.ShapeDtypeStruct((EP, H), jnp.float32),
                  jax.ShapeDtypeStruct((NC, NS, 16), jnp.float32),
                  jax.ShapeDtypeStruct((EP,), jnp.float32)],
        mesh=_mesh(),
        compiler_params=_cparams(),
        scratch_types=[
            pltpu.VMEM((3, C), jnp.int32),
            pltpu.VMEM((3, C), jnp.int32),
            pltpu.VMEM((2, C, H), jnp.float32),
            pltpu.VMEM((2, C, H), jnp.float32),
            pltpu.VMEM((2, C, H), jnp.float32),
            pltpu.VMEM((2, C), jnp.float32),
            pltpu.VMEM((16,), jnp.float32),
            pltpu.VMEM((H,), jnp.float32),
            pltpu.SemaphoreType.DMA,
            pltpu.SemaphoreType.DMA,
            pltpu.SemaphoreType.DMA,
            pltpu.SemaphoreType.DMA,
            pltpu.SemaphoreType.DMA,
        ],
    )


def _final_body(src_h, dst_h, g1_h, g2_h, dv_h, out_o,
                sidx3, didx3, g1b2, g2b2, dvb2, ob2,
                sem_a, sem_b, sem_d, sem_st, sem_ix):
    cid = lax.axis_index("c")
    sid = lax.axis_index("s")
    wid = cid * NS + sid
    e0 = wid * EPW
    pltpu.sync_copy(src_h.at[pl.ds(e0, C)], sidx3.at[0])
    pltpu.sync_copy(dst_h.at[pl.ds(e0, C)], didx3.at[0])
    pltpu.async_copy(g1_h.at[sidx3.at[0]], g1b2.at[0], sem_a)
    pltpu.async_copy(g2_h.at[didx3.at[0]], g2b2.at[0], sem_b)
    pltpu.async_copy(dv_h.at[pl.ds(e0, C)], dvb2.at[0], sem_d)
    pltpu.async_copy(src_h.at[pl.ds(e0 + C, C)], sidx3.at[1], sem_ix)
    pltpu.async_copy(dst_h.at[pl.ds(e0 + C, C)], didx3.at[1], sem_ix)

    def iter_body(ci, carry):
        cur = lax.rem(ci, 2)
        nxt = 1 - cur
        ix = lax.rem(ci, 3)
        base = e0 + ci * C
        pltpu.make_async_copy(g1_h.at[sidx3.at[ix]], g1b2.at[cur], sem_a).wait()
        pltpu.make_async_copy(g2_h.at[didx3.at[ix]], g2b2.at[cur], sem_b).wait()
        pltpu.make_async_copy(dv_h.at[pl.ds(base, C)], dvb2.at[cur], sem_d).wait()

        def group(g, c2):
            sl = pl.ds(g * 16, 16)
            z = g1b2[cur, sl] + g2b2[cur, sl] + dvb2[cur, sl]
            ob2[cur, sl] = 1.0 / (1.0 + jnp.exp(-z))
            return c2

        lax.fori_loop(0, C // 16, group, 0, unroll=False)
        pltpu.async_copy(ob2.at[cur], out_o.at[pl.ds(base, C)], sem_st)

        @pl.when(ci + 1 < NCHUNK)
        def _():
            nix = lax.rem(ci + 1, 3)
            nbase = base + C
            pltpu.make_async_copy(src_h.at[pl.ds(nbase, C)], sidx3.at[nix], sem_ix).wait()
            pltpu.make_async_copy(dst_h.at[pl.ds(nbase, C)], didx3.at[nix], sem_ix).wait()
            pltpu.async_copy(g1_h.at[sidx3.at[nix]], g1b2.at[nxt], sem_a)
            pltpu.async_copy(g2_h.at[didx3.at[nix]], g2b2.at[nxt], sem_b)

            @pl.when(ci > 0)
            def _():
                pltpu.make_async_copy(ob2.at[nxt], out_o.at[pl.ds(base - C, C)],
                                      sem_st).wait()

            pltpu.async_copy(dv_h.at[pl.ds(nbase, C)], dvb2.at[nxt], sem_d)

            @pl.when(ci + 2 < NCHUNK)
            def _():
                pix = lax.rem(ci + 2, 3)
                pbase = base + 2 * C
                pltpu.async_copy(src_h.at[pl.ds(pbase, C)], sidx3.at[pix], sem_ix)
                pltpu.async_copy(dst_h.at[pl.ds(pbase, C)], didx3.at[pix], sem_ix)

        return carry

    lax.fori_loop(0, NCHUNK, iter_body, 0, unroll=False)
    pltpu.make_async_copy(ob2.at[0], out_o.at[pl.ds(e0, C)], sem_st).wait()
    pltpu.make_async_copy(ob2.at[1], out_o.at[pl.ds(e0, C)], sem_st).wait()


@functools.cache
def _final_pass():
    return pl.kernel(
        _final_body,
        out_type=jax.ShapeDtypeStruct((EP,), jnp.float32),
        mesh=_mesh(),
        compiler_params=_cparams(),
        scratch_types=[
            pltpu.VMEM((3, C), jnp.int32),
            pltpu.VMEM((3, C), jnp.int32),
            pltpu.VMEM((2, C), jnp.float32),
            pltpu.VMEM((2, C), jnp.float32),
            pltpu.VMEM((2, C), jnp.float32),
            pltpu.VMEM((2, C), jnp.float32),
            pltpu.SemaphoreType.DMA,
            pltpu.SemaphoreType.DMA,
            pltpu.SemaphoreType.DMA,
            pltpu.SemaphoreType.DMA,
            pltpu.SemaphoreType.DMA,
        ],
    )


def _pad_nodes(x, width):
    out = jnp.zeros((NP, width), jnp.float32)
    return out.at[:N].set(x)


def _conv(h, src, dst, eaW, AW, p):
    q = h @ p["Wq"] + p["bq"]
    k = h @ p["Wk"] + p["bk"]
    v = h @ p["Wv"] + p["bv"]
    kv = _pad_nodes(jnp.concatenate([k, v], axis=1), 2 * H)
    K = jnp.max(jnp.linalg.norm(k, axis=1))
    s = jnp.linalg.norm(q, axis=1) * (K + AW) * RSQRT_H
    qpad = _pad_nodes(
        jnp.concatenate([q * RSQRT_H, -s[:, None],
                         jnp.zeros((N, HQ - H - 1), jnp.float32)], axis=1), HQ)
    z128 = jnp.zeros((NP, H), jnp.float32)
    z1 = jnp.zeros((NP,), jnp.float32)
    acc, den = _conv_edge_pass()(src, dst, qpad, kv, eaW, z128, z1)
    num = acc[0, :N] + acc[1, :N]
    denom = den[0, :N] + den[1, :N]
    out = jnp.where(denom[:, None] > 0, num / denom[:, None], 0.0)
    return out + h @ p["Ws"] + p["bs"]


def kernel(x, edge_attr, edge_index, params):
    p = params
    src = jnp.concatenate([edge_index[0], jnp.zeros((EP - E,), jnp.int32)])
    dst = jnp.concatenate([edge_index[1],
                           jnp.full((EP - E,), DPAD, jnp.int32)])
    eap = jnp.zeros((EP, 16), jnp.float32).at[:E].set(edge_attr)
    h = x @ p["node_W"] + p["node_b"]

    c1 = p["conv1"]
    eaW1 = eap @ (p["edge_W"] @ c1["We"]) + p["edge_b"] @ c1["We"]
    AW1 = jnp.max(jnp.linalg.norm(eaW1[:E], axis=1))
    h = jax.nn.leaky_relu(_conv(h, src, dst, eaW1, AW1, c1))

    w = p["eu3_W"] @ p["fc_W"]
    c = p["eu3_b"] @ p["fc_W"] + p["fc_b"]

    A1 = _pad_nodes(h @ p["eu1_W"][:H], H)
    B1 = _pad_nodes(h @ p["eu1_W"][H:2 * H], H)
    T1 = eap @ (p["edge_W"] @ p["eu1_W"][2 * H:]) + (
        p["edge_b"] @ p["eu1_W"][2 * H:] + p["eu1_b"])
    ea2, rm1, _ = _eu_pass()(src, dst, A1, B1, T1, jnp.zeros((H,), jnp.float32))
    Aea2 = jnp.sqrt(jnp.max(rm1))

    c2 = p["conv2"]
    eaW2 = ea2 @ c2["We"]
    AW2 = Aea2 * jnp.linalg.norm(c2["We"])
    h = jax.nn.leaky_relu(_conv(h, src, dst, eaW2, AW2, c2))

    A2 = _pad_nodes(h @ p["eu2_W"][:H], H)
    B2 = _pad_nodes(h @ p["eu2_W"][H:2 * H], H)
    T2 = ea2 @ p["eu2_W"][2 * H:] + p["eu2_b"]
    ea3, rm2, dvec = _eu_pass()(src, dst, A2, B2, T2, w[2 * H:, 0])
    Aea3 = jnp.sqrt(jnp.max(rm2))

    c3 = p["conv3"]
    eaW3 = ea3 @ c3["We"]
    AW3 = Aea3 * jnp.linalg.norm(c3["We"])
    h = _conv(h, src, dst, eaW3, AW3, c3)

    g1 = jnp.zeros((NP,), jnp.float32).at[:N].set((h @ w[:H])[:, 0])
    g2 = jnp.zeros((NP,), jnp.float32).at[:N].set((h @ w[H:2 * H])[:, 0] + c[0])
    out = _final_pass()(src, dst, g1, g2, dvec)
    return out[:E, None]
